# baseline (speedup 1.0000x reference)
import functools

import jax
import jax.numpy as jnp
from jax import lax
from jax.experimental import pallas as pl
from jax.experimental.pallas import tpu as pltpu
from jax.experimental.pallas import tpu_sc as plsc

K = 32
N = 4096
DF = 64
TW = 80
LEAKY = 0.1
BLKQ = 512
BLKN = 256
GCH = 128


def _prep_body(p1_ref, p2_ref, wa_ref, wb_ref, bm_ref, a1_ref, q2_ref):
    p1 = p1_ref[0]
    p2 = p2_ref[0]
    dn = (((0,), (0,)), ((), ()))
    a1 = lax.dot_general(p1, wa_ref[...], dn, preferred_element_type=jnp.float32)
    q2 = lax.dot_general(p2, wb_ref[...], dn, preferred_element_type=jnp.float32)
    a1_ref[0] = a1 + bm_ref[...]
    q2_ref[0] = q2


def _prep_call(points1, points2, waT, wbT, b_mlp):
    B = points1.shape[0]
    blk = 1024
    return pl.pallas_call(
        _prep_body,
        grid=(B, N // blk),
        in_specs=[
            pl.BlockSpec((1, DF, blk), lambda b, n: (b, 0, n)),
            pl.BlockSpec((1, DF, blk), lambda b, n: (b, 0, n)),
            pl.BlockSpec((DF, DF), lambda b, n: (0, 0)),
            pl.BlockSpec((DF, DF), lambda b, n: (0, 0)),
            pl.BlockSpec((1, DF), lambda b, n: (0, 0)),
        ],
        out_specs=[
            pl.BlockSpec((1, blk, DF), lambda b, n: (b, n, 0)),
            pl.BlockSpec((1, blk, DF), lambda b, n: (b, n, 0)),
        ],
        out_shape=[
            jax.ShapeDtypeStruct((B, N, DF), jnp.float32),
            jax.ShapeDtypeStruct((B, N, DF), jnp.float32),
        ],
    )(points1, points2, waT, wbT, b_mlp)


NCLS = 128
NGRP = N // NCLS
MAXKEY = 0x7FFFFFFF


def _pack_keys(dsl, g):
    b = lax.bitcast_convert_type(jnp.maximum(dsl, 0.0), jnp.int32)
    return (b & ~jnp.int32(NGRP - 1)) | g


def _knn_body(q_ref, c_ref, idx_ref, d_ref, a1_ref, a2_ref, a3_ref, a4_ref,
              a5_ref, a6_ref, dm_ref):
    t = pl.program_id(0)
    offset = (t % 2) * N
    q = q_ref[0]
    c = c_ref[0]
    s1 = jnp.sum(q * q, axis=0)
    s2 = jnp.sum(c * c, axis=0)
    dn = (((0,), (0,)), ((), ()))
    prod = lax.dot_general(c, q, dn, preferred_element_type=jnp.float32)
    d_ref[...] = (s1[None, :] + s2[:, None]) - 2.0 * prod

    mk = jnp.full((NCLS, BLKQ), MAXKEY, jnp.int32)
    arefs = (a1_ref, a2_ref, a3_ref, a4_ref, a5_ref, a6_ref)
    for r in arefs:
        r[...] = mk
    dm_ref[...] = mk

    def build(g, _):
        off = pl.multiple_of(g * NCLS, NCLS)
        v = _pack_keys(d_ref[pl.ds(off, NCLS), :], g)
        for r in arefs:
            a = r[...]
            lo = jnp.where(v < a, v, a)
            v = jnp.where(v < a, a, v)
            r[...] = lo
        dm_ref[...] = jnp.minimum(dm_ref[...], v)
        return 0

    lax.fori_loop(0, NGRP, build, 0)

    slane = lax.broadcasted_iota(jnp.int32, (NCLS, BLKQ), 0)
    big = jnp.int32(1 << 30)

    def extract(j, _):
        a1 = a1_ref[...]
        m = jnp.min(a1, axis=0, keepdims=True)
        eq = a1 == m
        fl = jnp.min(jnp.where(eq, slane, big), axis=0, keepdims=True)
        sel = eq & (slane == fl)
        col = ((m & jnp.int32(NGRP - 1)) << 7) | fl
        idx_ref[0, pl.ds(j, 1), :] = col + offset
        for ra, rb in zip(arefs[:-1], arefs[1:]):
            ra[...] = jnp.where(sel, rb[...], ra[...])
        a6_ref[...] = jnp.where(sel, MAXKEY, a6_ref[...])
        return m

    m31 = lax.fori_loop(0, K, extract, jnp.full((1, BLKQ), MAXKEY, jnp.int32))

    dmin = jnp.min(dm_ref[...], axis=0, keepdims=True)

    @pl.when(jnp.any(dmin < m31))
    def _fallback():
        colf = lax.broadcasted_iota(jnp.int32, (N, BLKQ), 0)
        inf = jnp.float32(jnp.inf)

        def body(j, _):
            d = d_ref[...]
            m = jnp.min(d, axis=0, keepdims=True)
            eqq = d == m
            am = jnp.min(jnp.where(eqq, colf, big), axis=0, keepdims=True)
            idx_ref[0, pl.ds(j, 1), :] = am + offset
            d_ref[...] = jnp.where(eqq & (colf == am), inf, d)
            return 0

        lax.fori_loop(0, K, body, 0)


def _knn_call(qs, cs):
    T = qs.shape[0]
    return pl.pallas_call(
        _knn_body,
        grid=(T, N // BLKQ),
        in_specs=[
            pl.BlockSpec((1, 3, BLKQ), lambda t, n: (t, 0, n)),
            pl.BlockSpec((1, 3, N), lambda t, n: (t, 0, 0)),
        ],
        out_specs=pl.BlockSpec((1, K, BLKQ), lambda t, n: (t, 0, n)),
        out_shape=jax.ShapeDtypeStruct((T, K, N), jnp.int32),
        scratch_shapes=[pltpu.VMEM((N, BLKQ), jnp.float32)] + [
            pltpu.VMEM((NCLS, BLKQ), jnp.int32) for _ in range(7)],
    )(qs, cs)


def _gather_call(table, idx):
    M = idx.shape[0]
    info = plsc.get_sparse_core_info()
    nw = info.num_cores * info.num_subcores
    per_w = M // nw
    mesh = plsc.VectorSubcoreMesh(core_axis_name="c", subcore_axis_name="s")

    nit = per_w // GCH

    @functools.partial(
        pl.kernel,
        mesh=mesh,
        out_type=jax.ShapeDtypeStruct((M, TW), jnp.float32),
        compiler_params=pltpu.CompilerParams(use_tc_tiling_on_sc=False),
        scratch_types=[
            pltpu.VMEM((per_w,), jnp.int32),
            pltpu.VMEM((2, GCH, TW), jnp.float32),
            pltpu.SemaphoreType.DMA,
            pltpu.SemaphoreType.DMA,
            pltpu.SemaphoreType.DMA,
            pltpu.SemaphoreType.DMA,
        ],
    )
    def gk(table_hbm, idx_hbm, out_hbm, idx_v, rows_v, g0, g1, w0, w1):
        wid = lax.axis_index("s") * info.num_cores + lax.axis_index("c")
        base = wid * per_w
        gsem = (g0, g1)
        wsem = (w0, w1)

        def gstart(i, b):
            pltpu.async_copy(table_hbm.at[idx_v.at[pl.ds(i * GCH, GCH)]],
                             rows_v.at[b], gsem[b])

        def gwait(b):
            pltpu.make_async_copy(table_hbm.at[idx_v.at[pl.ds(0, GCH)]],
                                  rows_v.at[b], gsem[b]).wait()

        def wstart(i, b):
            pltpu.async_copy(rows_v.at[b],
                             out_hbm.at[pl.ds(base + i * GCH, GCH)], wsem[b])

        def wwait(b):
            pltpu.make_async_copy(rows_v.at[b],
                                  out_hbm.at[pl.ds(base, GCH)], wsem[b]).wait()

        pltpu.sync_copy(idx_hbm.at[pl.ds(base, per_w)], idx_v)
        gstart(0, 0)

        def outer(io, _):
            i0 = io * 2
            for bs in range(2):
                i = i0 + bs
                nb = 1 - bs

                @pl.when(i >= 1)
                def _():
                    wwait(nb)

                @pl.when(i + 1 < nit)
                def _():
                    gstart(i + 1, nb)

                gwait(bs)
                wstart(i, bs)
            return 0

        lax.fori_loop(0, nit // 2, outer, 0)
        wwait(1)

    return gk(table, idx)


def _wnet3(d2, waT, ba, wbT, bb, wcT, bc):
    h = jnp.maximum(jnp.dot(d2, waT, preferred_element_type=jnp.float32) + ba, 0.0)
    h = jnp.maximum(jnp.dot(h, wbT, preferred_element_type=jnp.float32) + bb, 0.0)
    return jnp.maximum(jnp.dot(h, wcT, preferred_element_type=jnp.float32) + bc, 0.0)


def _stage1_body(g_ref, x1_ref, a1_ref, wc_ref, wa_ref, ba_ref, wb_ref,
                 bb_ref, wcc_ref, bc_ref, out_ref):
    g = g_ref[0]
    q2g = g[:, :, 0:DF]
    nxyz = g[:, :, DF:DF + 3]
    dirv = nxyz - x1_ref[0][None]
    d2 = dirv.reshape(K * BLKN, 3)
    dc = jnp.dot(d2, wc_ref[...], preferred_element_type=jnp.float32)
    cost = a1_ref[0][None] + q2g + dc.reshape(K, BLKN, DF)
    cost = jnp.where(cost >= 0.0, cost, LEAKY * cost)
    w = _wnet3(d2, wa_ref[...], ba_ref[...], wb_ref[...], bb_ref[...],
               wcc_ref[...], bc_ref[...])
    out_ref[0] = jnp.sum(w.reshape(K, BLKN, DF) * cost, axis=0)


def _stage2_body(g_ref, x1_ref, wa_ref, ba_ref, wb_ref, bb_ref, wc_ref,
                 bc_ref, out_ref):
    g = g_ref[0]
    pg = g[:, :, 0:DF]
    nxyz = g[:, :, DF:DF + 3]
    dirv = nxyz - x1_ref[0][None]
    d2 = dirv.reshape(K * BLKN, 3)
    w = _wnet3(d2, wa_ref[...], ba_ref[...], wb_ref[...], bb_ref[...],
               wc_ref[...], bc_ref[...])
    out_ref[0] = jnp.sum(w.reshape(K, BLKN, DF) * pg, axis=0)


def _small(shape):
    return pl.BlockSpec(shape, lambda b, n: tuple(0 for _ in shape))


def _stage1_call(g1, x1t, a1, wcT, w1aT, b1a, w1bT, b1b, w1cT, b1c):
    B = x1t.shape[0]
    return pl.pallas_call(
        _stage1_body,
        grid=(B, N // BLKN),
        in_specs=[
            pl.BlockSpec((1, K, BLKN, TW), lambda b, n: (b, 0, n, 0)),
            pl.BlockSpec((1, BLKN, 3), lambda b, n: (b, n, 0)),
            pl.BlockSpec((1, BLKN, DF), lambda b, n: (b, n, 0)),
            _small((3, DF)), _small((3, 8)), _small((1, 8)),
            _small((8, 8)), _small((1, 8)), _small((8, DF)), _small((1, DF)),
        ],
        out_specs=pl.BlockSpec((1, BLKN, DF), lambda b, n: (b, n, 0)),
        out_shape=jax.ShapeDtypeStruct((B, N, DF), jnp.float32),
    )(g1, x1t, a1, wcT, w1aT, b1a, w1bT, b1b, w1cT, b1c)


def _stage2_call(g2, x1t, w2aT, b2a, w2bT, b2b, w2cT, b2c):
    B = x1t.shape[0]
    return pl.pallas_call(
        _stage2_body,
        grid=(B, N // BLKN),
        in_specs=[
            pl.BlockSpec((1, K, BLKN, TW), lambda b, n: (b, 0, n, 0)),
            pl.BlockSpec((1, BLKN, 3), lambda b, n: (b, n, 0)),
            _small((3, 8)), _small((1, 8)),
            _small((8, 8)), _small((1, 8)), _small((8, DF)), _small((1, DF)),
        ],
        out_specs=pl.BlockSpec((1, BLKN, DF), lambda b, n: (b, n, 0)),
        out_shape=jax.ShapeDtypeStruct((B, N, DF), jnp.float32),
    )(g2, x1t, w2aT, b2a, w2bT, b2b, w2cT, b2c)


def kernel(xyz1, xyz2, points1, points2, W_mlp, b_mlp, w1a, b1a, w1b, b1b,
           w1c, b1c, w2a, b2a, w2b, b2b, w2c, b2c):
    B = xyz1.shape[0]
    x1t = jnp.transpose(xyz1, (0, 2, 1))
    x2t = jnp.transpose(xyz2, (0, 2, 1))

    waT = jnp.transpose(W_mlp[:, 0:DF])
    wbT = jnp.transpose(W_mlp[:, DF:2 * DF])
    wcT = jnp.transpose(W_mlp[:, 2 * DF:])

    a1, q2 = _prep_call(points1, points2, waT, wbT, b_mlp[None, :])

    idx1 = _knn_call(xyz1, xyz2).reshape(-1)
    idx2 = _knn_call(xyz1, xyz1).reshape(-1)

    pad = jnp.zeros((B, N, TW - DF - 3), jnp.float32)
    table1 = jnp.concatenate([q2, x2t, pad], axis=-1).reshape(B * N, TW)
    g1 = _gather_call(table1, idx1).reshape(B, K, N, TW)

    p2p = _stage1_call(g1, x1t, a1, wcT,
                       jnp.transpose(w1a), b1a[None, :],
                       jnp.transpose(w1b), b1b[None, :],
                       jnp.transpose(w1c), b1c[None, :])

    table2 = jnp.concatenate([p2p, x1t, pad], axis=-1).reshape(B * N, TW)
    g2 = _gather_call(table2, idx2).reshape(B, K, N, TW)

    out = _stage2_call(g2, x1t,
                       jnp.transpose(w2a), b2a[None, :],
                       jnp.transpose(w2b), b2b[None, :],
                       jnp.transpose(w2c), b2c[None, :])
    return jnp.transpose(out, (0, 2, 1))


# speedup vs baseline: 1.2209x; 1.2209x over previous
import functools

import jax
import jax.numpy as jnp
from jax import lax
from jax.experimental import pallas as pl
from jax.experimental.pallas import tpu as pltpu
from jax.experimental.pallas import tpu_sc as plsc

K = 32
N = 4096
DF = 64
TW = 128
LEAKY = 0.1
BLKQ = 512
BLKN = 256
GCH = 128


def _prep_body(p1_ref, p2_ref, wa_ref, wb_ref, bm_ref, a1_ref, q2_ref):
    p1 = p1_ref[0]
    p2 = p2_ref[0]
    dn = (((0,), (0,)), ((), ()))
    a1 = lax.dot_general(p1, wa_ref[...], dn, preferred_element_type=jnp.float32)
    q2 = lax.dot_general(p2, wb_ref[...], dn, preferred_element_type=jnp.float32)
    a1_ref[0] = a1 + bm_ref[...]
    q2_ref[0] = q2


def _prep_call(points1, points2, waT, wbT, b_mlp):
    B = points1.shape[0]
    blk = 1024
    return pl.pallas_call(
        _prep_body,
        grid=(B, N // blk),
        in_specs=[
            pl.BlockSpec((1, DF, blk), lambda b, n: (b, 0, n)),
            pl.BlockSpec((1, DF, blk), lambda b, n: (b, 0, n)),
            pl.BlockSpec((DF, DF), lambda b, n: (0, 0)),
            pl.BlockSpec((DF, DF), lambda b, n: (0, 0)),
            pl.BlockSpec((1, DF), lambda b, n: (0, 0)),
        ],
        out_specs=[
            pl.BlockSpec((1, blk, DF), lambda b, n: (b, n, 0)),
            pl.BlockSpec((1, blk, DF), lambda b, n: (b, n, 0)),
        ],
        out_shape=[
            jax.ShapeDtypeStruct((B, N, DF), jnp.float32),
            jax.ShapeDtypeStruct((B, N, DF), jnp.float32),
        ],
    )(points1, points2, waT, wbT, b_mlp)


NCLS = 128
NGRP = N // NCLS
MAXKEY = 0x7FFFFFFF


def _pack_keys(dsl, g):
    b = lax.bitcast_convert_type(jnp.maximum(dsl, 0.0), jnp.int32)
    return (b & ~jnp.int32(NGRP - 1)) | g


def _knn_body(q_ref, c_ref, idx_ref, d_ref, a1_ref, a2_ref, a3_ref, a4_ref,
              a5_ref, a6_ref, dm_ref):
    t = pl.program_id(0)
    offset = (t % 2) * N
    q = q_ref[0]
    c = c_ref[0]
    s1 = jnp.sum(q * q, axis=0)
    s2 = jnp.sum(c * c, axis=0)
    dn = (((0,), (0,)), ((), ()))
    prod = lax.dot_general(c, q, dn, preferred_element_type=jnp.float32)
    d_ref[...] = (s1[None, :] + s2[:, None]) - 2.0 * prod

    mk = jnp.full((NCLS, BLKQ), MAXKEY, jnp.int32)
    arefs = (a1_ref, a2_ref, a3_ref, a4_ref, a5_ref, a6_ref)
    for r in arefs:
        r[...] = mk
    dm_ref[...] = mk

    def build(g, _):
        off = pl.multiple_of(g * NCLS, NCLS)
        v = _pack_keys(d_ref[pl.ds(off, NCLS), :], g)
        for r in arefs:
            a = r[...]
            lo = jnp.where(v < a, v, a)
            v = jnp.where(v < a, a, v)
            r[...] = lo
        dm_ref[...] = jnp.minimum(dm_ref[...], v)
        return 0

    lax.fori_loop(0, NGRP, build, 0)

    slane = lax.broadcasted_iota(jnp.int32, (NCLS, BLKQ), 0)
    big = jnp.int32(1 << 30)

    def extract(j, _):
        a1 = a1_ref[...]
        m = jnp.min(a1, axis=0, keepdims=True)
        eq = a1 == m
        fl = jnp.min(jnp.where(eq, slane, big), axis=0, keepdims=True)
        sel = eq & (slane == fl)
        col = ((m & jnp.int32(NGRP - 1)) << 7) | fl
        idx_ref[0, pl.ds(j, 1), :] = col + offset
        for ra, rb in zip(arefs[:-1], arefs[1:]):
            ra[...] = jnp.where(sel, rb[...], ra[...])
        a6_ref[...] = jnp.where(sel, MAXKEY, a6_ref[...])
        return m

    m31 = lax.fori_loop(0, K, extract, jnp.full((1, BLKQ), MAXKEY, jnp.int32))

    dmin = jnp.min(dm_ref[...], axis=0, keepdims=True)

    @pl.when(jnp.any(dmin < m31))
    def _fallback():
        colf = lax.broadcasted_iota(jnp.int32, (N, BLKQ), 0)
        inf = jnp.float32(jnp.inf)

        def body(j, _):
            d = d_ref[...]
            m = jnp.min(d, axis=0, keepdims=True)
            eqq = d == m
            am = jnp.min(jnp.where(eqq, colf, big), axis=0, keepdims=True)
            idx_ref[0, pl.ds(j, 1), :] = am + offset
            d_ref[...] = jnp.where(eqq & (colf == am), inf, d)
            return 0

        lax.fori_loop(0, K, body, 0)


def _knn_call(qs, cs):
    T = qs.shape[0]
    return pl.pallas_call(
        _knn_body,
        grid=(T, N // BLKQ),
        in_specs=[
            pl.BlockSpec((1, 3, BLKQ), lambda t, n: (t, 0, n)),
            pl.BlockSpec((1, 3, N), lambda t, n: (t, 0, 0)),
        ],
        out_specs=pl.BlockSpec((1, K, BLKQ), lambda t, n: (t, 0, n)),
        out_shape=jax.ShapeDtypeStruct((T, K, N), jnp.int32),
        scratch_shapes=[pltpu.VMEM((N, BLKQ), jnp.float32)] + [
            pltpu.VMEM((NCLS, BLKQ), jnp.int32) for _ in range(7)],
    )(qs, cs)


def _gather_call(table, idx):
    M = idx.shape[0]
    info = plsc.get_sparse_core_info()
    nw = info.num_cores * info.num_subcores
    per_w = M // nw
    mesh = plsc.VectorSubcoreMesh(core_axis_name="c", subcore_axis_name="s")

    nit = per_w // GCH

    @functools.partial(
        pl.kernel,
        mesh=mesh,
        out_type=jax.ShapeDtypeStruct((M, TW), jnp.float32),
        scratch_types=[
            pltpu.VMEM((per_w,), jnp.int32),
            pltpu.VMEM((2, GCH, TW), jnp.float32),
            pltpu.SemaphoreType.DMA,
            pltpu.SemaphoreType.DMA,
            pltpu.SemaphoreType.DMA,
            pltpu.SemaphoreType.DMA,
        ],
    )
    def gk(table_hbm, idx_hbm, out_hbm, idx_v, rows_v, g0, g1, w0, w1):
        wid = lax.axis_index("s") * info.num_cores + lax.axis_index("c")
        base = wid * per_w
        gsem = (g0, g1)
        wsem = (w0, w1)

        def gstart(i, b):
            pltpu.async_copy(table_hbm.at[idx_v.at[pl.ds(i * GCH, GCH)]],
                             rows_v.at[b], gsem[b])

        def gwait(b):
            pltpu.make_async_copy(table_hbm.at[idx_v.at[pl.ds(0, GCH)]],
                                  rows_v.at[b], gsem[b]).wait()

        def wstart(i, b):
            pltpu.async_copy(rows_v.at[b],
                             out_hbm.at[pl.ds(base + i * GCH, GCH)], wsem[b])

        def wwait(b):
            pltpu.make_async_copy(rows_v.at[b],
                                  out_hbm.at[pl.ds(base, GCH)], wsem[b]).wait()

        pltpu.sync_copy(idx_hbm.at[pl.ds(base, per_w)], idx_v)
        gstart(0, 0)

        def outer(io, _):
            i0 = io * 2
            for bs in range(2):
                i = i0 + bs
                nb = 1 - bs

                @pl.when(i >= 1)
                def _():
                    wwait(nb)

                @pl.when(i + 1 < nit)
                def _():
                    gstart(i + 1, nb)

                gwait(bs)
                wstart(i, bs)
            return 0

        lax.fori_loop(0, nit // 2, outer, 0)
        wwait(1)

    return gk(table, idx)


def _wnet3(d2, waT, ba, wbT, bb, wcT, bc):
    h = jnp.maximum(jnp.dot(d2, waT, preferred_element_type=jnp.float32) + ba, 0.0)
    h = jnp.maximum(jnp.dot(h, wbT, preferred_element_type=jnp.float32) + bb, 0.0)
    return jnp.maximum(jnp.dot(h, wcT, preferred_element_type=jnp.float32) + bc, 0.0)


def _stage1_body(g_ref, x1_ref, a1_ref, wc_ref, wa_ref, ba_ref, wb_ref,
                 bb_ref, wcc_ref, bc_ref, out_ref):
    g = g_ref[0]
    q2g = g[:, :, 0:DF]
    nxyz = g[:, :, DF:DF + 3]
    dirv = nxyz - x1_ref[0][None]
    d2 = dirv.reshape(K * BLKN, 3)
    dc = jnp.dot(d2, wc_ref[...], preferred_element_type=jnp.float32)
    cost = a1_ref[0][None] + q2g + dc.reshape(K, BLKN, DF)
    cost = jnp.where(cost >= 0.0, cost, LEAKY * cost)
    w = _wnet3(d2, wa_ref[...], ba_ref[...], wb_ref[...], bb_ref[...],
               wcc_ref[...], bc_ref[...])
    out_ref[0] = jnp.sum(w.reshape(K, BLKN, DF) * cost, axis=0)


def _stage2_body(g_ref, x1_ref, wa_ref, ba_ref, wb_ref, bb_ref, wc_ref,
                 bc_ref, out_ref):
    g = g_ref[0]
    pg = g[:, :, 0:DF]
    nxyz = g[:, :, DF:DF + 3]
    dirv = nxyz - x1_ref[0][None]
    d2 = dirv.reshape(K * BLKN, 3)
    w = _wnet3(d2, wa_ref[...], ba_ref[...], wb_ref[...], bb_ref[...],
               wc_ref[...], bc_ref[...])
    out_ref[0] = jnp.sum(w.reshape(K, BLKN, DF) * pg, axis=0)


def _small(shape):
    return pl.BlockSpec(shape, lambda b, n: tuple(0 for _ in shape))


def _stage1_call(g1, x1t, a1, wcT, w1aT, b1a, w1bT, b1b, w1cT, b1c):
    B = x1t.shape[0]
    return pl.pallas_call(
        _stage1_body,
        grid=(B, N // BLKN),
        in_specs=[
            pl.BlockSpec((1, K, BLKN, TW), lambda b, n: (b, 0, n, 0)),
            pl.BlockSpec((1, BLKN, 3), lambda b, n: (b, n, 0)),
            pl.BlockSpec((1, BLKN, DF), lambda b, n: (b, n, 0)),
            _small((3, DF)), _small((3, 8)), _small((1, 8)),
            _small((8, 8)), _small((1, 8)), _small((8, DF)), _small((1, DF)),
        ],
        out_specs=pl.BlockSpec((1, BLKN, DF), lambda b, n: (b, n, 0)),
        out_shape=jax.ShapeDtypeStruct((B, N, DF), jnp.float32),
    )(g1, x1t, a1, wcT, w1aT, b1a, w1bT, b1b, w1cT, b1c)


def _stage2_call(g2, x1t, w2aT, b2a, w2bT, b2b, w2cT, b2c):
    B = x1t.shape[0]
    return pl.pallas_call(
        _stage2_body,
        grid=(B, N // BLKN),
        in_specs=[
            pl.BlockSpec((1, K, BLKN, TW), lambda b, n: (b, 0, n, 0)),
            pl.BlockSpec((1, BLKN, 3), lambda b, n: (b, n, 0)),
            _small((3, 8)), _small((1, 8)),
            _small((8, 8)), _small((1, 8)), _small((8, DF)), _small((1, DF)),
        ],
        out_specs=pl.BlockSpec((1, BLKN, DF), lambda b, n: (b, n, 0)),
        out_shape=jax.ShapeDtypeStruct((B, N, DF), jnp.float32),
    )(g2, x1t, w2aT, b2a, w2bT, b2b, w2cT, b2c)


def kernel(xyz1, xyz2, points1, points2, W_mlp, b_mlp, w1a, b1a, w1b, b1b,
           w1c, b1c, w2a, b2a, w2b, b2b, w2c, b2c):
    B = xyz1.shape[0]
    x1t = jnp.transpose(xyz1, (0, 2, 1))
    x2t = jnp.transpose(xyz2, (0, 2, 1))

    waT = jnp.transpose(W_mlp[:, 0:DF])
    wbT = jnp.transpose(W_mlp[:, DF:2 * DF])
    wcT = jnp.transpose(W_mlp[:, 2 * DF:])

    a1, q2 = _prep_call(points1, points2, waT, wbT, b_mlp[None, :])

    idx1 = _knn_call(xyz1, xyz2).reshape(-1)
    idx2 = _knn_call(xyz1, xyz1).reshape(-1)

    pad = jnp.zeros((B, N, TW - DF - 3), jnp.float32)
    table1 = jnp.concatenate([q2, x2t, pad], axis=-1).reshape(B * N, TW)
    g1 = _gather_call(table1, idx1).reshape(B, K, N, TW)

    p2p = _stage1_call(g1, x1t, a1, wcT,
                       jnp.transpose(w1a), b1a[None, :],
                       jnp.transpose(w1b), b1b[None, :],
                       jnp.transpose(w1c), b1c[None, :])

    table2 = jnp.concatenate([p2p, x1t, pad], axis=-1).reshape(B * N, TW)
    g2 = _gather_call(table2, idx2).reshape(B, K, N, TW)

    out = _stage2_call(g2, x1t,
                       jnp.transpose(w2a), b2a[None, :],
                       jnp.transpose(w2b), b2b[None, :],
                       jnp.transpose(w2c), b2c[None, :])
    return jnp.transpose(out, (0, 2, 1))


# speedup vs baseline: 1.2368x; 1.0130x over previous
import functools

import jax
import jax.numpy as jnp
from jax import lax
from jax.experimental import pallas as pl
from jax.experimental.pallas import tpu as pltpu
from jax.experimental.pallas import tpu_sc as plsc

K = 32
N = 4096
DF = 64
TW = 128
LEAKY = 0.1
BLKQ = 1024
BLKN = 512
GCH = 128


def _prep_body(p1_ref, p2_ref, wa_ref, wb_ref, bm_ref, a1_ref, q2_ref):
    p1 = p1_ref[0]
    p2 = p2_ref[0]
    dn = (((0,), (0,)), ((), ()))
    a1 = lax.dot_general(p1, wa_ref[...], dn, preferred_element_type=jnp.float32)
    q2 = lax.dot_general(p2, wb_ref[...], dn, preferred_element_type=jnp.float32)
    a1_ref[0] = a1 + bm_ref[...]
    q2_ref[0] = q2


def _prep_call(points1, points2, waT, wbT, b_mlp):
    B = points1.shape[0]
    blk = 1024
    return pl.pallas_call(
        _prep_body,
        grid=(B, N // blk),
        in_specs=[
            pl.BlockSpec((1, DF, blk), lambda b, n: (b, 0, n)),
            pl.BlockSpec((1, DF, blk), lambda b, n: (b, 0, n)),
            pl.BlockSpec((DF, DF), lambda b, n: (0, 0)),
            pl.BlockSpec((DF, DF), lambda b, n: (0, 0)),
            pl.BlockSpec((1, DF), lambda b, n: (0, 0)),
        ],
        out_specs=[
            pl.BlockSpec((1, blk, DF), lambda b, n: (b, n, 0)),
            pl.BlockSpec((1, blk, DF), lambda b, n: (b, n, 0)),
        ],
        out_shape=[
            jax.ShapeDtypeStruct((B, N, DF), jnp.float32),
            jax.ShapeDtypeStruct((B, N, DF), jnp.float32),
        ],
    )(points1, points2, waT, wbT, b_mlp)


NCLS = 128
NGRP = N // NCLS
MAXKEY = 0x7FFFFFFF


def _pack_keys(dsl, g):
    b = lax.bitcast_convert_type(jnp.maximum(dsl, 0.0), jnp.int32)
    return (b & ~jnp.int32(NGRP - 1)) | g


def _knn_body(q_ref, c_ref, idx_ref, d_ref, a1_ref, a2_ref, a3_ref, a4_ref,
              a5_ref, a6_ref, dm_ref):
    t = pl.program_id(0)
    offset = (t % 2) * N
    q = q_ref[0]
    c = c_ref[0]
    s1 = jnp.sum(q * q, axis=0)
    s2 = jnp.sum(c * c, axis=0)
    dn = (((0,), (0,)), ((), ()))
    prod = lax.dot_general(c, q, dn, preferred_element_type=jnp.float32)
    d_ref[...] = (s1[None, :] + s2[:, None]) - 2.0 * prod

    mk = jnp.full((NCLS, BLKQ), MAXKEY, jnp.int32)
    arefs = (a1_ref, a2_ref, a3_ref, a4_ref, a5_ref, a6_ref)
    for r in arefs:
        r[...] = mk
    dm_ref[...] = mk

    def build(g, _):
        off = pl.multiple_of(g * NCLS, NCLS)
        v = _pack_keys(d_ref[pl.ds(off, NCLS), :], g)
        for r in arefs:
            a = r[...]
            lo = jnp.where(v < a, v, a)
            v = jnp.where(v < a, a, v)
            r[...] = lo
        dm_ref[...] = jnp.minimum(dm_ref[...], v)
        return 0

    lax.fori_loop(0, NGRP, build, 0)

    slane = lax.broadcasted_iota(jnp.int32, (NCLS, BLKQ), 0)
    big = jnp.int32(1 << 30)

    def extract(j, _):
        a1 = a1_ref[...]
        m = jnp.min(a1, axis=0, keepdims=True)
        eq = a1 == m
        fl = jnp.min(jnp.where(eq, slane, big), axis=0, keepdims=True)
        sel = eq & (slane == fl)
        col = ((m & jnp.int32(NGRP - 1)) << 7) | fl
        idx_ref[0, pl.ds(j, 1), :] = col + offset
        for ra, rb in zip(arefs[:-1], arefs[1:]):
            ra[...] = jnp.where(sel, rb[...], ra[...])
        a6_ref[...] = jnp.where(sel, MAXKEY, a6_ref[...])
        return m

    m31 = lax.fori_loop(0, K, extract, jnp.full((1, BLKQ), MAXKEY, jnp.int32))

    dmin = jnp.min(dm_ref[...], axis=0, keepdims=True)

    @pl.when(jnp.any(dmin < m31))
    def _fallback():
        colf = lax.broadcasted_iota(jnp.int32, (N, BLKQ), 0)
        inf = jnp.float32(jnp.inf)

        def body(j, _):
            d = d_ref[...]
            m = jnp.min(d, axis=0, keepdims=True)
            eqq = d == m
            am = jnp.min(jnp.where(eqq, colf, big), axis=0, keepdims=True)
            idx_ref[0, pl.ds(j, 1), :] = am + offset
            d_ref[...] = jnp.where(eqq & (colf == am), inf, d)
            return 0

        lax.fori_loop(0, K, body, 0)


def _knn_call(qs, cs):
    T = qs.shape[0]
    return pl.pallas_call(
        _knn_body,
        grid=(T, N // BLKQ),
        in_specs=[
            pl.BlockSpec((1, 3, BLKQ), lambda t, n: (t, 0, n)),
            pl.BlockSpec((1, 3, N), lambda t, n: (t, 0, 0)),
        ],
        out_specs=pl.BlockSpec((1, K, BLKQ), lambda t, n: (t, 0, n)),
        out_shape=jax.ShapeDtypeStruct((T, K, N), jnp.int32),
        scratch_shapes=[pltpu.VMEM((N, BLKQ), jnp.float32)] + [
            pltpu.VMEM((NCLS, BLKQ), jnp.int32) for _ in range(7)],
    )(qs, cs)


def _gather_call(table, idx):
    M = idx.shape[0]
    info = plsc.get_sparse_core_info()
    nw = info.num_cores * info.num_subcores
    per_w = M // nw
    mesh = plsc.VectorSubcoreMesh(core_axis_name="c", subcore_axis_name="s")

    nit = per_w // GCH

    @functools.partial(
        pl.kernel,
        mesh=mesh,
        out_type=jax.ShapeDtypeStruct((M, TW), jnp.float32),
        scratch_types=[
            pltpu.VMEM((per_w,), jnp.int32),
            pltpu.VMEM((2, GCH, TW), jnp.float32),
            pltpu.SemaphoreType.DMA,
            pltpu.SemaphoreType.DMA,
            pltpu.SemaphoreType.DMA,
            pltpu.SemaphoreType.DMA,
        ],
    )
    def gk(table_hbm, idx_hbm, out_hbm, idx_v, rows_v, g0, g1, w0, w1):
        wid = lax.axis_index("s") * info.num_cores + lax.axis_index("c")
        base = wid * per_w
        gsem = (g0, g1)
        wsem = (w0, w1)

        def gstart(i, b):
            pltpu.async_copy(table_hbm.at[idx_v.at[pl.ds(i * GCH, GCH)]],
                             rows_v.at[b], gsem[b])

        def gwait(b):
            pltpu.make_async_copy(table_hbm.at[idx_v.at[pl.ds(0, GCH)]],
                                  rows_v.at[b], gsem[b]).wait()

        def wstart(i, b):
            pltpu.async_copy(rows_v.at[b],
                             out_hbm.at[pl.ds(base + i * GCH, GCH)], wsem[b])

        def wwait(b):
            pltpu.make_async_copy(rows_v.at[b],
                                  out_hbm.at[pl.ds(base, GCH)], wsem[b]).wait()

        pltpu.sync_copy(idx_hbm.at[pl.ds(base, per_w)], idx_v)
        gstart(0, 0)

        def outer(io, _):
            i0 = io * 2
            for bs in range(2):
                i = i0 + bs
                nb = 1 - bs

                @pl.when(i >= 1)
                def _():
                    wwait(nb)

                @pl.when(i + 1 < nit)
                def _():
                    gstart(i + 1, nb)

                gwait(bs)
                wstart(i, bs)
            return 0

        lax.fori_loop(0, nit // 2, outer, 0)
        wwait(1)

    return gk(table, idx)


def _wnet3(d2, waT, ba, wbT, bb, wcT, bc):
    h = jnp.maximum(jnp.dot(d2, waT, preferred_element_type=jnp.float32) + ba, 0.0)
    h = jnp.maximum(jnp.dot(h, wbT, preferred_element_type=jnp.float32) + bb, 0.0)
    return jnp.maximum(jnp.dot(h, wcT, preferred_element_type=jnp.float32) + bc, 0.0)


def _stage1_body(g_ref, x1_ref, a1_ref, wc_ref, wa_ref, ba_ref, wb_ref,
                 bb_ref, wcc_ref, bc_ref, out_ref):
    g = g_ref[0]
    q2g = g[:, :, 0:DF]
    nxyz = g[:, :, DF:DF + 3]
    dirv = nxyz - x1_ref[0][None]
    d2 = dirv.reshape(K * BLKN, 3)
    dc = jnp.dot(d2, wc_ref[...], preferred_element_type=jnp.float32)
    cost = a1_ref[0][None] + q2g + dc.reshape(K, BLKN, DF)
    cost = jnp.where(cost >= 0.0, cost, LEAKY * cost)
    w = _wnet3(d2, wa_ref[...], ba_ref[...], wb_ref[...], bb_ref[...],
               wcc_ref[...], bc_ref[...])
    out_ref[0] = jnp.sum(w.reshape(K, BLKN, DF) * cost, axis=0)


def _stage2_body(g_ref, x1_ref, wa_ref, ba_ref, wb_ref, bb_ref, wc_ref,
                 bc_ref, out_ref):
    g = g_ref[0]
    pg = g[:, :, 0:DF]
    nxyz = g[:, :, DF:DF + 3]
    dirv = nxyz - x1_ref[0][None]
    d2 = dirv.reshape(K * BLKN, 3)
    w = _wnet3(d2, wa_ref[...], ba_ref[...], wb_ref[...], bb_ref[...],
               wc_ref[...], bc_ref[...])
    out_ref[0] = jnp.sum(w.reshape(K, BLKN, DF) * pg, axis=0)


def _small(shape):
    return pl.BlockSpec(shape, lambda b, n: tuple(0 for _ in shape))


def _stage1_call(g1, x1t, a1, wcT, w1aT, b1a, w1bT, b1b, w1cT, b1c):
    B = x1t.shape[0]
    return pl.pallas_call(
        _stage1_body,
        grid=(B, N // BLKN),
        in_specs=[
            pl.BlockSpec((1, K, BLKN, TW), lambda b, n: (b, 0, n, 0)),
            pl.BlockSpec((1, BLKN, 3), lambda b, n: (b, n, 0)),
            pl.BlockSpec((1, BLKN, DF), lambda b, n: (b, n, 0)),
            _small((3, DF)), _small((3, 8)), _small((1, 8)),
            _small((8, 8)), _small((1, 8)), _small((8, DF)), _small((1, DF)),
        ],
        out_specs=pl.BlockSpec((1, BLKN, DF), lambda b, n: (b, n, 0)),
        out_shape=jax.ShapeDtypeStruct((B, N, DF), jnp.float32),
    )(g1, x1t, a1, wcT, w1aT, b1a, w1bT, b1b, w1cT, b1c)


def _stage2_call(g2, x1t, w2aT, b2a, w2bT, b2b, w2cT, b2c):
    B = x1t.shape[0]
    return pl.pallas_call(
        _stage2_body,
        grid=(B, N // BLKN),
        in_specs=[
            pl.BlockSpec((1, K, BLKN, TW), lambda b, n: (b, 0, n, 0)),
            pl.BlockSpec((1, BLKN, 3), lambda b, n: (b, n, 0)),
            _small((3, 8)), _small((1, 8)),
            _small((8, 8)), _small((1, 8)), _small((8, DF)), _small((1, DF)),
        ],
        out_specs=pl.BlockSpec((1, BLKN, DF), lambda b, n: (b, n, 0)),
        out_shape=jax.ShapeDtypeStruct((B, N, DF), jnp.float32),
    )(g2, x1t, w2aT, b2a, w2bT, b2b, w2cT, b2c)


def kernel(xyz1, xyz2, points1, points2, W_mlp, b_mlp, w1a, b1a, w1b, b1b,
           w1c, b1c, w2a, b2a, w2b, b2b, w2c, b2c):
    B = xyz1.shape[0]
    x1t = jnp.transpose(xyz1, (0, 2, 1))
    x2t = jnp.transpose(xyz2, (0, 2, 1))

    waT = jnp.transpose(W_mlp[:, 0:DF])
    wbT = jnp.transpose(W_mlp[:, DF:2 * DF])
    wcT = jnp.transpose(W_mlp[:, 2 * DF:])

    a1, q2 = _prep_call(points1, points2, waT, wbT, b_mlp[None, :])

    idx1 = _knn_call(xyz1, xyz2).reshape(-1)
    idx2 = _knn_call(xyz1, xyz1).reshape(-1)

    pad = jnp.zeros((B, N, TW - DF - 3), jnp.float32)
    table1 = jnp.concatenate([q2, x2t, pad], axis=-1).reshape(B * N, TW)
    g1 = _gather_call(table1, idx1).reshape(B, K, N, TW)

    p2p = _stage1_call(g1, x1t, a1, wcT,
                       jnp.transpose(w1a), b1a[None, :],
                       jnp.transpose(w1b), b1b[None, :],
                       jnp.transpose(w1c), b1c[None, :])

    table2 = jnp.concatenate([p2p, x1t, pad], axis=-1).reshape(B * N, TW)
    g2 = _gather_call(table2, idx2).reshape(B, K, N, TW)

    out = _stage2_call(g2, x1t,
                       jnp.transpose(w2a), b2a[None, :],
                       jnp.transpose(w2b), b2b[None, :],
                       jnp.transpose(w2c), b2c[None, :])
    return jnp.transpose(out, (0, 2, 1))


# speedup vs baseline: 1.2486x; 1.0095x over previous
import functools

import jax
import jax.numpy as jnp
from jax import lax
from jax.experimental import pallas as pl
from jax.experimental.pallas import tpu as pltpu
from jax.experimental.pallas import tpu_sc as plsc

K = 32
N = 4096
DF = 64
TW = 128
LEAKY = 0.1
BLKQ = 1024
BLKN = 512
GCH = 128


def _prep_body(p1_ref, p2_ref, wa_ref, wb_ref, bm_ref, a1_ref, q2_ref):
    p1 = p1_ref[0]
    p2 = p2_ref[0]
    dn = (((0,), (0,)), ((), ()))
    a1 = lax.dot_general(p1, wa_ref[...], dn, preferred_element_type=jnp.float32)
    q2 = lax.dot_general(p2, wb_ref[...], dn, preferred_element_type=jnp.float32)
    a1_ref[0] = a1 + bm_ref[...]
    q2_ref[0] = q2


def _prep_call(points1, points2, waT, wbT, b_mlp):
    B = points1.shape[0]
    blk = 1024
    return pl.pallas_call(
        _prep_body,
        grid=(B, N // blk),
        in_specs=[
            pl.BlockSpec((1, DF, blk), lambda b, n: (b, 0, n)),
            pl.BlockSpec((1, DF, blk), lambda b, n: (b, 0, n)),
            pl.BlockSpec((DF, DF), lambda b, n: (0, 0)),
            pl.BlockSpec((DF, DF), lambda b, n: (0, 0)),
            pl.BlockSpec((1, DF), lambda b, n: (0, 0)),
        ],
        out_specs=[
            pl.BlockSpec((1, blk, DF), lambda b, n: (b, n, 0)),
            pl.BlockSpec((1, blk, DF), lambda b, n: (b, n, 0)),
        ],
        out_shape=[
            jax.ShapeDtypeStruct((B, N, DF), jnp.float32),
            jax.ShapeDtypeStruct((B, N, DF), jnp.float32),
        ],
    )(points1, points2, waT, wbT, b_mlp)


NCLS = 64
NGRP = N // NCLS
NDEPTH = 8
MAXKEY = 0x7FFFFFFF


def _pack_keys(dsl, g):
    b = lax.bitcast_convert_type(jnp.maximum(dsl, 0.0), jnp.int32)
    return (b & ~jnp.int32(NGRP - 1)) | g


def _knn_body(q_ref, c_ref, idx_ref, d_ref, *scr):
    arefs, dm_ref = scr[:NDEPTH], scr[NDEPTH]
    t = pl.program_id(0)
    offset = (t % 2) * N
    q = q_ref[0]
    c = c_ref[0]
    s1 = jnp.sum(q * q, axis=0)
    s2 = jnp.sum(c * c, axis=0)
    dn = (((0,), (0,)), ((), ()))
    prod = lax.dot_general(c, q, dn, preferred_element_type=jnp.float32)
    d_ref[...] = (s1[None, :] + s2[:, None]) - 2.0 * prod

    mk = jnp.full((NCLS, BLKQ), MAXKEY, jnp.int32)
    for r in arefs:
        r[...] = mk
    dm_ref[...] = mk

    def build(g, _):
        off = pl.multiple_of(g * NCLS, NCLS)
        v = _pack_keys(d_ref[pl.ds(off, NCLS), :], g)
        for r in arefs:
            a = r[...]
            lo = jnp.where(v < a, v, a)
            v = jnp.where(v < a, a, v)
            r[...] = lo
        dm_ref[...] = jnp.minimum(dm_ref[...], v)
        return 0

    lax.fori_loop(0, NGRP, build, 0)

    slane = lax.broadcasted_iota(jnp.int32, (NCLS, BLKQ), 0)
    big = jnp.int32(1 << 30)

    def extract(j, _):
        a1 = arefs[0][...]
        m = jnp.min(a1, axis=0, keepdims=True)
        eq = a1 == m
        fl = jnp.min(jnp.where(eq, slane, big), axis=0, keepdims=True)
        sel = eq & (slane == fl)
        col = ((m & jnp.int32(NGRP - 1)) << 6) | fl
        idx_ref[0, pl.ds(j, 1), :] = col + offset
        for ra, rb in zip(arefs[:-1], arefs[1:]):
            ra[...] = jnp.where(sel, rb[...], ra[...])
        arefs[-1][...] = jnp.where(sel, MAXKEY, arefs[-1][...])
        return m

    m31 = lax.fori_loop(0, K, extract, jnp.full((1, BLKQ), MAXKEY, jnp.int32))

    dmin = jnp.min(dm_ref[...], axis=0, keepdims=True)

    @pl.when(jnp.any(dmin < m31))
    def _fallback():
        colf = lax.broadcasted_iota(jnp.int32, (N, BLKQ), 0)
        inf = jnp.float32(jnp.inf)

        def body(j, _):
            d = d_ref[...]
            m = jnp.min(d, axis=0, keepdims=True)
            eqq = d == m
            am = jnp.min(jnp.where(eqq, colf, big), axis=0, keepdims=True)
            idx_ref[0, pl.ds(j, 1), :] = am + offset
            d_ref[...] = jnp.where(eqq & (colf == am), inf, d)
            return 0

        lax.fori_loop(0, K, body, 0)


def _knn_call(qs, cs):
    T = qs.shape[0]
    return pl.pallas_call(
        _knn_body,
        grid=(T, N // BLKQ),
        in_specs=[
            pl.BlockSpec((1, 3, BLKQ), lambda t, n: (t, 0, n)),
            pl.BlockSpec((1, 3, N), lambda t, n: (t, 0, 0)),
        ],
        out_specs=pl.BlockSpec((1, K, BLKQ), lambda t, n: (t, 0, n)),
        out_shape=jax.ShapeDtypeStruct((T, K, N), jnp.int32),
        scratch_shapes=[pltpu.VMEM((N, BLKQ), jnp.float32)] + [
            pltpu.VMEM((NCLS, BLKQ), jnp.int32) for _ in range(NDEPTH + 1)],
    )(qs, cs)


def _gather_call(table, idx):
    M = idx.shape[0]
    info = plsc.get_sparse_core_info()
    nw = info.num_cores * info.num_subcores
    per_w = M // nw
    mesh = plsc.VectorSubcoreMesh(core_axis_name="c", subcore_axis_name="s")

    nit = per_w // GCH

    @functools.partial(
        pl.kernel,
        mesh=mesh,
        out_type=jax.ShapeDtypeStruct((M, TW), jnp.float32),
        scratch_types=[
            pltpu.VMEM((per_w,), jnp.int32),
            pltpu.VMEM((2, GCH, TW), jnp.float32),
            pltpu.SemaphoreType.DMA,
            pltpu.SemaphoreType.DMA,
            pltpu.SemaphoreType.DMA,
            pltpu.SemaphoreType.DMA,
        ],
    )
    def gk(table_hbm, idx_hbm, out_hbm, idx_v, rows_v, g0, g1, w0, w1):
        wid = lax.axis_index("s") * info.num_cores + lax.axis_index("c")
        base = wid * per_w
        gsem = (g0, g1)
        wsem = (w0, w1)

        def gstart(i, b):
            pltpu.async_copy(table_hbm.at[idx_v.at[pl.ds(i * GCH, GCH)]],
                             rows_v.at[b], gsem[b])

        def gwait(b):
            pltpu.make_async_copy(table_hbm.at[idx_v.at[pl.ds(0, GCH)]],
                                  rows_v.at[b], gsem[b]).wait()

        def wstart(i, b):
            pltpu.async_copy(rows_v.at[b],
                             out_hbm.at[pl.ds(base + i * GCH, GCH)], wsem[b])

        def wwait(b):
            pltpu.make_async_copy(rows_v.at[b],
                                  out_hbm.at[pl.ds(base, GCH)], wsem[b]).wait()

        pltpu.sync_copy(idx_hbm.at[pl.ds(base, per_w)], idx_v)
        gstart(0, 0)

        def outer(io, _):
            i0 = io * 2
            for bs in range(2):
                i = i0 + bs
                nb = 1 - bs

                @pl.when(i >= 1)
                def _():
                    wwait(nb)

                @pl.when(i + 1 < nit)
                def _():
                    gstart(i + 1, nb)

                gwait(bs)
                wstart(i, bs)
            return 0

        lax.fori_loop(0, nit // 2, outer, 0)
        wwait(1)

    return gk(table, idx)


def _wnet3(d2, waT, ba, wbT, bb, wcT, bc):
    h = jnp.maximum(jnp.dot(d2, waT, preferred_element_type=jnp.float32) + ba, 0.0)
    h = jnp.maximum(jnp.dot(h, wbT, preferred_element_type=jnp.float32) + bb, 0.0)
    return jnp.maximum(jnp.dot(h, wcT, preferred_element_type=jnp.float32) + bc, 0.0)


def _stage1_body(g_ref, x1_ref, a1_ref, wc_ref, wa_ref, ba_ref, wb_ref,
                 bb_ref, wcc_ref, bc_ref, out_ref):
    g = g_ref[0]
    q2g = g[:, :, 0:DF]
    nxyz = g[:, :, DF:DF + 3]
    dirv = nxyz - x1_ref[0][None]
    d2 = dirv.reshape(K * BLKN, 3)
    dc = jnp.dot(d2, wc_ref[...], preferred_element_type=jnp.float32)
    cost = a1_ref[0][None] + q2g + dc.reshape(K, BLKN, DF)
    cost = jnp.where(cost >= 0.0, cost, LEAKY * cost)
    w = _wnet3(d2, wa_ref[...], ba_ref[...], wb_ref[...], bb_ref[...],
               wcc_ref[...], bc_ref[...])
    out_ref[0] = jnp.sum(w.reshape(K, BLKN, DF) * cost, axis=0)


def _stage2_body(g_ref, x1_ref, wa_ref, ba_ref, wb_ref, bb_ref, wc_ref,
                 bc_ref, out_ref):
    g = g_ref[0]
    pg = g[:, :, 0:DF]
    nxyz = g[:, :, DF:DF + 3]
    dirv = nxyz - x1_ref[0][None]
    d2 = dirv.reshape(K * BLKN, 3)
    w = _wnet3(d2, wa_ref[...], ba_ref[...], wb_ref[...], bb_ref[...],
               wc_ref[...], bc_ref[...])
    out_ref[0] = jnp.sum(w.reshape(K, BLKN, DF) * pg, axis=0)


def _small(shape):
    return pl.BlockSpec(shape, lambda b, n: tuple(0 for _ in shape))


def _stage1_call(g1, x1t, a1, wcT, w1aT, b1a, w1bT, b1b, w1cT, b1c):
    B = x1t.shape[0]
    return pl.pallas_call(
        _stage1_body,
        grid=(B, N // BLKN),
        in_specs=[
            pl.BlockSpec((1, K, BLKN, TW), lambda b, n: (b, 0, n, 0)),
            pl.BlockSpec((1, BLKN, 3), lambda b, n: (b, n, 0)),
            pl.BlockSpec((1, BLKN, DF), lambda b, n: (b, n, 0)),
            _small((3, DF)), _small((3, 8)), _small((1, 8)),
            _small((8, 8)), _small((1, 8)), _small((8, DF)), _small((1, DF)),
        ],
        out_specs=pl.BlockSpec((1, BLKN, DF), lambda b, n: (b, n, 0)),
        out_shape=jax.ShapeDtypeStruct((B, N, DF), jnp.float32),
    )(g1, x1t, a1, wcT, w1aT, b1a, w1bT, b1b, w1cT, b1c)


def _stage2_call(g2, x1t, w2aT, b2a, w2bT, b2b, w2cT, b2c):
    B = x1t.shape[0]
    return pl.pallas_call(
        _stage2_body,
        grid=(B, N // BLKN),
        in_specs=[
            pl.BlockSpec((1, K, BLKN, TW), lambda b, n: (b, 0, n, 0)),
            pl.BlockSpec((1, BLKN, 3), lambda b, n: (b, n, 0)),
            _small((3, 8)), _small((1, 8)),
            _small((8, 8)), _small((1, 8)), _small((8, DF)), _small((1, DF)),
        ],
        out_specs=pl.BlockSpec((1, BLKN, DF), lambda b, n: (b, n, 0)),
        out_shape=jax.ShapeDtypeStruct((B, N, DF), jnp.float32),
    )(g2, x1t, w2aT, b2a, w2bT, b2b, w2cT, b2c)


def kernel(xyz1, xyz2, points1, points2, W_mlp, b_mlp, w1a, b1a, w1b, b1b,
           w1c, b1c, w2a, b2a, w2b, b2b, w2c, b2c):
    B = xyz1.shape[0]
    x1t = jnp.transpose(xyz1, (0, 2, 1))
    x2t = jnp.transpose(xyz2, (0, 2, 1))

    waT = jnp.transpose(W_mlp[:, 0:DF])
    wbT = jnp.transpose(W_mlp[:, DF:2 * DF])
    wcT = jnp.transpose(W_mlp[:, 2 * DF:])

    a1, q2 = _prep_call(points1, points2, waT, wbT, b_mlp[None, :])

    idx1 = _knn_call(xyz1, xyz2).reshape(-1)
    idx2 = _knn_call(xyz1, xyz1).reshape(-1)

    pad = jnp.zeros((B, N, TW - DF - 3), jnp.float32)
    table1 = jnp.concatenate([q2, x2t, pad], axis=-1).reshape(B * N, TW)
    g1 = _gather_call(table1, idx1).reshape(B, K, N, TW)

    p2p = _stage1_call(g1, x1t, a1, wcT,
                       jnp.transpose(w1a), b1a[None, :],
                       jnp.transpose(w1b), b1b[None, :],
                       jnp.transpose(w1c), b1c[None, :])

    table2 = jnp.concatenate([p2p, x1t, pad], axis=-1).reshape(B * N, TW)
    g2 = _gather_call(table2, idx2).reshape(B, K, N, TW)

    out = _stage2_call(g2, x1t,
                       jnp.transpose(w2a), b2a[None, :],
                       jnp.transpose(w2b), b2b[None, :],
                       jnp.transpose(w2c), b2c[None, :])
    return jnp.transpose(out, (0, 2, 1))
